# Initial kernel scaffold; baseline (speedup 1.0000x reference)
import jax
import jax.numpy as jnp
from jax.experimental import pallas as pl


def kernel(mesh_signal, bary_coordinates, neighbor_weights, self_weights, bias):
    raise NotImplementedError("write your pallas kernel here")


# speedup vs baseline: 4.7317x; 4.7317x over previous
import functools

import jax
import jax.numpy as jnp
from jax import lax
from jax.experimental import pallas as pl
from jax.experimental.pallas import tpu as pltpu
from jax.experimental.pallas import tpu_sc as plsc

N = 10000
R = 5
A = 8
F = 128
T = 8
NROT = 4
RA = R * A
CH = NROT * T
G = R * A * 3
GP = 128

NC, NS = 2, 16
NW = NC * NS
VB = 8
NB = 40
NP = NW * VB * NB


def _tc_project(ms_pad, wroll, wc, bias32):
    BLK = 512

    def body(ms_ref, wr_ref, wc_ref, b_ref, q_ref, c_ref):
        x = ms_ref[...]
        q_ref[...] = jnp.dot(x, wr_ref[...], preferred_element_type=jnp.float32)
        c_ref[...] = jnp.dot(x, wc_ref[...], preferred_element_type=jnp.float32) + b_ref[...]

    return pl.pallas_call(
        body,
        grid=(NP // BLK,),
        in_specs=[
            pl.BlockSpec((BLK, F), lambda i: (i, 0)),
            pl.BlockSpec((F, RA * CH), lambda i: (0, 0)),
            pl.BlockSpec((F, CH), lambda i: (0, 0)),
            pl.BlockSpec((1, CH), lambda i: (0, 0)),
        ],
        out_specs=[
            pl.BlockSpec((BLK, RA * CH), lambda i: (i, 0)),
            pl.BlockSpec((BLK, CH), lambda i: (i, 0)),
        ],
        out_shape=[
            jax.ShapeDtypeStruct((NP, RA * CH), jnp.float32),
            jax.ShapeDtypeStruct((NP, CH), jnp.float32),
        ],
    )(ms_pad, wroll, wc, bias32)


def _sc_gather_accum(table, idx_flat, w_flat, c32_flat, offs):
    mesh = plsc.VectorSubcoreMesh(
        core_axis_name="c", subcore_axis_name="s", num_cores=NC, num_subcores=NS
    )

    @functools.partial(
        pl.kernel,
        out_type=jax.ShapeDtypeStruct((NP * CH,), jnp.float32),
        mesh=mesh,
        scratch_types=[
            pltpu.VMEM((GP,), jnp.int32),
            pltpu.VMEM((VB * GP,), jnp.int32),
            pltpu.VMEM((VB * GP,), jnp.int32),
            pltpu.VMEM((VB * GP,), jnp.float32),
            pltpu.VMEM((VB * GP, CH), jnp.float32),
            pltpu.VMEM((VB * CH,), jnp.float32),
            pltpu.VMEM((VB * CH,), jnp.float32),
            pltpu.SemaphoreType.DMA,
        ],
        compiler_params=pltpu.CompilerParams(
            needs_layout_passes=False, use_tc_tiling_on_sc=False
        ),
    )
    def k(table_h, idx_h, w_h, c32_h, offs_h, out_h,
          offs_v, idx_v, row_v, w_v, gath_v, c32_v, out_v, sem):
        wid = lax.axis_index("s") * NC + lax.axis_index("c")
        pltpu.sync_copy(offs_h, offs_v)
        base0 = wid * (NB * VB)

        def block_body(b, carry):
            vbase = base0 + b * VB
            pltpu.sync_copy(idx_h.at[pl.ds(vbase * GP, VB * GP)], idx_v)
            pltpu.sync_copy(w_h.at[pl.ds(vbase * GP, VB * GP)], w_v)
            pltpu.sync_copy(c32_h.at[pl.ds(vbase * CH, VB * CH)], c32_v)
            for s in range(VB * GP // 16):
                row_v[pl.ds(s * 16, 16)] = (
                    idx_v[pl.ds(s * 16, 16)] * RA
                    + offs_v[pl.ds((s % (GP // 16)) * 16, 16)]
                )
            pltpu.async_copy(table_h.at[row_v], gath_v, sem).wait()

            def vert(p, c2):
                acc0 = c32_v[pl.ds(p * CH, 16)]
                acc1 = c32_v[pl.ds(p * CH + 16, 16)]
                for i in range(G):
                    pos = p * GP + i
                    wi = plsc.load_gather(
                        w_v, [jnp.full((16,), pos, jnp.int32)]
                    )
                    acc0 = acc0 + wi * gath_v[pos, pl.ds(0, 16)]
                    acc1 = acc1 + wi * gath_v[pos, pl.ds(16, 16)]
                out_v[pl.ds(p * CH, 16)] = jnp.maximum(acc0, 0.0)
                out_v[pl.ds(p * CH + 16, 16)] = jnp.maximum(acc1, 0.0)
                return c2

            lax.fori_loop(0, VB, vert, 0)
            pltpu.sync_copy(out_v, out_h.at[pl.ds(vbase * CH, VB * CH)])
            return carry

        lax.fori_loop(0, NB, block_body, 0)

    return k(table, idx_flat, w_flat, c32_flat, offs)


@jax.jit
def kernel(mesh_signal, bary_coordinates, neighbor_weights, self_weights, bias):
    rolled = jnp.stack(
        [jnp.roll(neighbor_weights, -2 * oi, axis=2) for oi in range(NROT)], axis=0
    )
    wroll = rolled.transpose(2, 3, 0, 1, 4).reshape(RA * CH, F).T
    wc = jnp.tile(self_weights[:, 0, :].T, (1, NROT))
    bias32 = jnp.tile(bias, NROT)[None, :]

    ms_pad = jnp.pad(mesh_signal, ((0, NP - N), (0, 0)))

    idx = bary_coordinates[..., 0].astype(jnp.int32).reshape(N, G)
    w = bary_coordinates[..., 1].reshape(N, G)
    idx_pad = jnp.pad(idx, ((0, NP - N), (0, GP - G))).reshape(NP * GP)
    w_pad = jnp.pad(w, ((0, NP - N), (0, GP - G))).reshape(NP * GP)
    offs = jnp.pad(jnp.arange(G, dtype=jnp.int32) // 3, (0, GP - G))

    qroll, c32 = _tc_project(ms_pad, wroll, wc, bias32)
    table = qroll.reshape(NP * RA, CH)

    out = _sc_gather_accum(table, idx_pad, w_pad, c32.reshape(NP * CH), offs)

    return out.reshape(NP, NROT, T)[:N]


# speedup vs baseline: 4.8027x; 1.0150x over previous
import functools

import jax
import jax.numpy as jnp
from jax import lax
from jax.experimental import pallas as pl
from jax.experimental.pallas import tpu as pltpu
from jax.experimental.pallas import tpu_sc as plsc

N = 10000
R = 5
A = 8
F = 128
T = 8
NROT = 4
RA = R * A
CH = NROT * T
G = R * A * 3
GP = 128

NC, NS = 2, 16
NW = NC * NS
VB = 8
NB = 40
NH = NB // 2
NP = NW * VB * NB

SV = GP + GP + CH
SB = VB * SV
W_OFF = VB * GP
C_OFF = 2 * VB * GP


def _tc_project(ms_pad, wroll, wc, bias32):
    BLK = 512

    def body(ms_ref, wr_ref, wc_ref, b_ref, q_ref, c_ref):
        x = ms_ref[...]
        q_ref[...] = jnp.dot(x, wr_ref[...], preferred_element_type=jnp.float32)
        c_ref[...] = jnp.dot(x, wc_ref[...], preferred_element_type=jnp.float32) + b_ref[...]

    return pl.pallas_call(
        body,
        grid=(NP // BLK,),
        in_specs=[
            pl.BlockSpec((BLK, F), lambda i: (i, 0)),
            pl.BlockSpec((F, RA * CH), lambda i: (0, 0)),
            pl.BlockSpec((F, CH), lambda i: (0, 0)),
            pl.BlockSpec((1, CH), lambda i: (0, 0)),
        ],
        out_specs=[
            pl.BlockSpec((BLK, RA * CH), lambda i: (i, 0)),
            pl.BlockSpec((BLK, CH), lambda i: (i, 0)),
        ],
        out_shape=[
            jax.ShapeDtypeStruct((NP, RA * CH), jnp.float32),
            jax.ShapeDtypeStruct((NP, CH), jnp.float32),
        ],
    )(ms_pad, wroll, wc, bias32)


def _sc_gather_accum(table, idx_flat, w_flat, c32_flat, offs):
    mesh = plsc.VectorSubcoreMesh(
        core_axis_name="c", subcore_axis_name="s", num_cores=NC, num_subcores=NS
    )

    @functools.partial(
        pl.kernel,
        out_type=jax.ShapeDtypeStruct((NP * CH,), jnp.float32),
        mesh=mesh,
        scratch_types=[
            pltpu.VMEM((GP,), jnp.int32),
            pltpu.VMEM((NH * VB * GP,), jnp.int32),
            pltpu.VMEM((NH * VB * GP,), jnp.float32),
            pltpu.VMEM((NH * VB * CH,), jnp.float32),
            pltpu.VMEM((VB * GP,), jnp.int32),
            pltpu.VMEM((VB * GP,), jnp.int32),
            pltpu.VMEM((VB * GP, CH), jnp.float32),
            pltpu.VMEM((VB * GP, CH), jnp.float32),
            pltpu.VMEM((NH * VB * CH,), jnp.float32),
            pltpu.SemaphoreType.DMA,
            pltpu.SemaphoreType.DMA,
        ],
        compiler_params=pltpu.CompilerParams(
            needs_layout_passes=False, use_tc_tiling_on_sc=False
        ),
    )
    def k(table_h, idx_h, w_h, c32_h, offs_h, out_h,
          offs_v, sidx_v, sw_v, sc32_v, row0_v, row1_v, gath0_v, gath1_v,
          out_v, sem_g0, sem_g1):
        wid = lax.axis_index("s") * NC + lax.axis_index("c")
        pltpu.sync_copy(offs_h, offs_v)
        base0 = wid * NB
        slots = ((row0_v, gath0_v, sem_g0), (row1_v, gath1_v, sem_g1))

        def rows(sb, par):
            row_v = slots[par][0]
            for s in range(VB * GP // 16):
                row_v[pl.ds(s * 16, 16)] = (
                    sidx_v[pl.ds(sb * (VB * GP) + s * 16, 16)] * RA
                    + offs_v[pl.ds((s % (GP // 16)) * 16, 16)]
                )

        def start_gather(par):
            row_v, gath_v, sem = slots[par]
            return pltpu.async_copy(table_h.at[row_v], gath_v, sem)

        def wait_gather(par):
            row_v, gath_v, sem = slots[par]
            pltpu.make_async_copy(table_h.at[row_v], gath_v, sem).wait()

        def compute(sb, par):
            gath_v = slots[par][1]

            def vert(p, c2):
                cbase = sb * (VB * CH) + p * CH
                acc0 = sc32_v[pl.ds(cbase, 16)]
                acc1 = sc32_v[pl.ds(cbase + 16, 16)]
                wbase = sb * (VB * GP) + p * GP
                for i in range(G):
                    wi = plsc.load_gather(
                        sw_v, [jnp.full((16,), wbase + i, jnp.int32)]
                    )
                    pos = p * GP + i
                    acc0 = acc0 + wi * gath_v[pos, pl.ds(0, 16)]
                    acc1 = acc1 + wi * gath_v[pos, pl.ds(16, 16)]
                obase = sb * (VB * CH) + p * CH
                out_v[pl.ds(obase, 16)] = jnp.maximum(acc0, 0.0)
                out_v[pl.ds(obase + 16, 16)] = jnp.maximum(acc1, 0.0)
                return c2

            lax.fori_loop(0, VB, vert, 0)

        def half(h, carry):
            hbase = base0 + h * NH
            pltpu.sync_copy(
                idx_h.at[pl.ds(hbase * (VB * GP), NH * VB * GP)], sidx_v
            )
            pltpu.sync_copy(
                w_h.at[pl.ds(hbase * (VB * GP), NH * VB * GP)], sw_v
            )
            pltpu.sync_copy(
                c32_h.at[pl.ds(hbase * (VB * CH), NH * VB * CH)], sc32_v
            )
            rows(0, 0)
            start_gather(0)

            def pair(t, c2):
                sb0 = 2 * t
                rows(sb0 + 1, 1)
                start_gather(1)
                wait_gather(0)
                compute(sb0, 0)

                @pl.when(t < NH // 2 - 1)
                def _():
                    rows(sb0 + 2, 0)
                    start_gather(0)

                wait_gather(1)
                compute(sb0 + 1, 1)
                return c2

            lax.fori_loop(0, NH // 2, pair, 0)
            pltpu.sync_copy(
                out_v, out_h.at[pl.ds(hbase * (VB * CH), NH * VB * CH)]
            )
            return carry

        lax.fori_loop(0, 2, half, 0)

    return k(table, idx_flat, w_flat, c32_flat, offs)


@jax.jit
def kernel(mesh_signal, bary_coordinates, neighbor_weights, self_weights, bias):
    rolled = jnp.stack(
        [jnp.roll(neighbor_weights, -2 * oi, axis=2) for oi in range(NROT)], axis=0
    )
    wroll = rolled.transpose(2, 3, 0, 1, 4).reshape(RA * CH, F).T
    wc = jnp.tile(self_weights[:, 0, :].T, (1, NROT))
    bias32 = jnp.tile(bias, NROT)[None, :]

    ms_pad = jnp.pad(mesh_signal, ((0, NP - N), (0, 0)))

    idx = bary_coordinates[..., 0].astype(jnp.int32).reshape(N, G)
    w = bary_coordinates[..., 1].reshape(N, G)
    idx_pad = jnp.pad(idx, ((0, NP - N), (0, GP - G))).reshape(NP // VB, VB * GP)
    w_pad = jnp.pad(w, ((0, NP - N), (0, GP - G))).reshape(NP // VB, VB * GP)
    offs = jnp.pad(jnp.arange(G, dtype=jnp.int32) // 3, (0, GP - G))

    qroll, c32 = _tc_project(ms_pad, wroll, wc, bias32)
    table = qroll.reshape(NP * RA, CH)

    out = _sc_gather_accum(
        table,
        idx_pad.reshape(-1),
        w_pad.reshape(-1),
        c32.reshape(-1),
        offs,
    )

    return out.reshape(NP, NROT, T)[:N]


# speedup vs baseline: 7.3063x; 1.5213x over previous
import functools

import jax
import jax.numpy as jnp
from jax import lax
from jax.experimental import pallas as pl
from jax.experimental.pallas import tpu as pltpu
from jax.experimental.pallas import tpu_sc as plsc

N = 10000
R = 5
A = 8
F = 128
T = 8
NROT = 4
RA = R * A
CH = NROT * T
G = R * A * 3
GP = 128

NC, NS = 2, 16
NW = NC * NS
VB = 8
NB = 40
NH = NB // 2
NP = NW * VB * NB

SV = GP + GP + CH
SB = VB * SV
W_OFF = VB * GP
C_OFF = 2 * VB * GP


def _tc_project(ms_pad, wroll, wc, bias32):
    BLK = 512

    def body(ms_ref, wr_ref, wc_ref, b_ref, q_ref, c_ref):
        x = ms_ref[...]
        q_ref[...] = jnp.dot(
            x, wr_ref[...], preferred_element_type=jnp.float32
        ).astype(jnp.bfloat16)
        c_ref[...] = jnp.dot(x, wc_ref[...], preferred_element_type=jnp.float32) + b_ref[...]

    return pl.pallas_call(
        body,
        grid=(NP // BLK,),
        in_specs=[
            pl.BlockSpec((BLK, F), lambda i: (i, 0)),
            pl.BlockSpec((F, RA * CH), lambda i: (0, 0)),
            pl.BlockSpec((F, CH), lambda i: (0, 0)),
            pl.BlockSpec((1, CH), lambda i: (0, 0)),
        ],
        out_specs=[
            pl.BlockSpec((BLK, RA * CH), lambda i: (i, 0)),
            pl.BlockSpec((BLK, CH), lambda i: (i, 0)),
        ],
        out_shape=[
            jax.ShapeDtypeStruct((NP, RA * CH), jnp.bfloat16),
            jax.ShapeDtypeStruct((NP, CH), jnp.float32),
        ],
    )(ms_pad, wroll, wc, bias32)


def _sc_gather_accum(table, idx_flat, w_flat, c32_flat, offs):
    mesh = plsc.VectorSubcoreMesh(
        core_axis_name="c", subcore_axis_name="s", num_cores=NC, num_subcores=NS
    )

    @functools.partial(
        pl.kernel,
        out_type=jax.ShapeDtypeStruct((NP * CH,), jnp.float32),
        mesh=mesh,
        scratch_types=[
            pltpu.VMEM((GP,), jnp.int32),
            pltpu.VMEM((NH * VB * GP,), jnp.int32),
            pltpu.VMEM((NH * VB * GP,), jnp.float32),
            pltpu.VMEM((NH * VB * CH,), jnp.float32),
            pltpu.VMEM((VB * GP,), jnp.int32),
            pltpu.VMEM((VB * GP,), jnp.int32),
            pltpu.VMEM((VB * GP, CH), jnp.bfloat16),
            pltpu.VMEM((VB * GP, CH), jnp.bfloat16),
            pltpu.VMEM((NH * VB * CH,), jnp.float32),
            pltpu.SemaphoreType.DMA,
            pltpu.SemaphoreType.DMA,
        ],
        compiler_params=pltpu.CompilerParams(
            needs_layout_passes=False, use_tc_tiling_on_sc=False
        ),
    )
    def k(table_h, idx_h, w_h, c32_h, offs_h, out_h,
          offs_v, sidx_v, sw_v, sc32_v, row0_v, row1_v, gath0_v, gath1_v,
          out_v, sem_g0, sem_g1):
        wid = lax.axis_index("s") * NC + lax.axis_index("c")
        pltpu.sync_copy(offs_h, offs_v)
        base0 = wid * NB
        slots = ((row0_v, gath0_v, sem_g0), (row1_v, gath1_v, sem_g1))

        def rows(sb, par):
            row_v = slots[par][0]
            for s in range(VB * GP // 16):
                row_v[pl.ds(s * 16, 16)] = (
                    sidx_v[pl.ds(sb * (VB * GP) + s * 16, 16)] * RA
                    + offs_v[pl.ds((s % (GP // 16)) * 16, 16)]
                )

        def start_gather(par):
            row_v, gath_v, sem = slots[par]
            return pltpu.async_copy(table_h.at[row_v], gath_v, sem)

        def wait_gather(par):
            row_v, gath_v, sem = slots[par]
            pltpu.make_async_copy(table_h.at[row_v], gath_v, sem).wait()

        def compute(sb, par):
            gath_v = slots[par][1]

            def vert(p, c2):
                cbase = sb * (VB * CH) + p * CH
                acc0 = sc32_v[pl.ds(cbase, 16)]
                acc1 = sc32_v[pl.ds(cbase + 16, 16)]
                wbase = sb * (VB * GP) + p * GP
                for i in range(G):
                    wi = plsc.load_gather(
                        sw_v, [jnp.full((16,), wbase + i, jnp.int32)]
                    )
                    pos = p * GP + i
                    g0, g1 = plsc.unpack(
                        gath_v[pos, pl.ds(0, CH)],
                        format=plsc.PackFormat.INTERLEAVED,
                    )
                    acc0 = acc0 + wi * g0
                    acc1 = acc1 + wi * g1
                obase = sb * (VB * CH) + p * CH
                out_v[pl.ds(obase, 16)] = jnp.maximum(acc0, 0.0)
                out_v[pl.ds(obase + 16, 16)] = jnp.maximum(acc1, 0.0)
                return c2

            lax.fori_loop(0, VB, vert, 0)

        def half(h, carry):
            hbase = base0 + h * NH
            pltpu.sync_copy(
                idx_h.at[pl.ds(hbase * (VB * GP), NH * VB * GP)], sidx_v
            )
            pltpu.sync_copy(
                w_h.at[pl.ds(hbase * (VB * GP), NH * VB * GP)], sw_v
            )
            pltpu.sync_copy(
                c32_h.at[pl.ds(hbase * (VB * CH), NH * VB * CH)], sc32_v
            )
            rows(0, 0)
            start_gather(0)

            def pair(t, c2):
                sb0 = 2 * t
                rows(sb0 + 1, 1)
                start_gather(1)
                wait_gather(0)
                compute(sb0, 0)

                @pl.when(t < NH // 2 - 1)
                def _():
                    rows(sb0 + 2, 0)
                    start_gather(0)

                wait_gather(1)
                compute(sb0 + 1, 1)
                return c2

            lax.fori_loop(0, NH // 2, pair, 0)
            pltpu.sync_copy(
                out_v, out_h.at[pl.ds(hbase * (VB * CH), NH * VB * CH)]
            )
            return carry

        lax.fori_loop(0, 2, half, 0)

    return k(table, idx_flat, w_flat, c32_flat, offs)


@jax.jit
def kernel(mesh_signal, bary_coordinates, neighbor_weights, self_weights, bias):
    rolled = jnp.stack(
        [jnp.roll(neighbor_weights, -2 * oi, axis=2) for oi in range(NROT)], axis=0
    )
    wroll = (
        rolled.transpose(2, 3, 0, 1, 4)
        .reshape(RA, 2, CH // 2, F)
        .transpose(0, 2, 1, 3)
        .reshape(RA * CH, F)
        .T
    )
    wc = jnp.tile(self_weights[:, 0, :].T, (1, NROT))
    bias32 = jnp.tile(bias, NROT)[None, :]

    ms_pad = jnp.pad(mesh_signal, ((0, NP - N), (0, 0)))

    idx = bary_coordinates[..., 0].astype(jnp.int32).reshape(N, G)
    w = bary_coordinates[..., 1].reshape(N, G)
    idx_pad = jnp.pad(idx, ((0, NP - N), (0, GP - G))).reshape(NP // VB, VB * GP)
    w_pad = jnp.pad(w, ((0, NP - N), (0, GP - G))).reshape(NP // VB, VB * GP)
    offs = jnp.pad(jnp.arange(G, dtype=jnp.int32) // 3, (0, GP - G))

    qroll, c32 = _tc_project(ms_pad, wroll, wc, bias32)
    table = qroll.reshape(NP * RA, CH)

    out = _sc_gather_accum(
        table,
        idx_pad.reshape(-1),
        w_pad.reshape(-1),
        c32.reshape(-1),
        offs,
    )

    return out.reshape(NP, NROT, T)[:N]


# speedup vs baseline: 14.8065x; 2.0265x over previous
import functools

import jax
import jax.numpy as jnp
from jax import lax
from jax.experimental import pallas as pl
from jax.experimental.pallas import tpu as pltpu
from jax.experimental.pallas import tpu_sc as plsc

N = 10000
R = 5
A = 8
F = 128
T = 8
NROT = 4
RA = R * A
CH = NROT * T
G = R * A * 3
GP = 128

NC, NS = 2, 16
NW = NC * NS
VB = 8
NB = 40
NH = NB // 2
NP = NW * VB * NB

SV = GP + GP + CH
SB = VB * SV
W_OFF = VB * GP
C_OFF = 2 * VB * GP


def _tc_project(ms_pad, wroll, wc, bias32):
    BLK = 512

    def body(ms_ref, wr_ref, wc_ref, b_ref, q_ref, c_ref):
        x = ms_ref[...]
        q_ref[...] = jnp.dot(
            x, wr_ref[...], preferred_element_type=jnp.float32
        ).astype(jnp.bfloat16)
        c_ref[...] = jnp.dot(x, wc_ref[...], preferred_element_type=jnp.float32) + b_ref[...]

    return pl.pallas_call(
        body,
        grid=(NP // BLK,),
        in_specs=[
            pl.BlockSpec((BLK, F), lambda i: (i, 0)),
            pl.BlockSpec((F, RA * CH), lambda i: (0, 0)),
            pl.BlockSpec((F, CH), lambda i: (0, 0)),
            pl.BlockSpec((1, CH), lambda i: (0, 0)),
        ],
        out_specs=[
            pl.BlockSpec((BLK, RA * CH), lambda i: (i, 0)),
            pl.BlockSpec((BLK, CH), lambda i: (i, 0)),
        ],
        out_shape=[
            jax.ShapeDtypeStruct((NP, RA * CH), jnp.bfloat16),
            jax.ShapeDtypeStruct((NP, CH), jnp.float32),
        ],
    )(ms_pad, wroll, wc, bias32)


def _sc_gather_accum(table, idx_flat, w_flat, c32_flat, offs):
    mesh = plsc.VectorSubcoreMesh(
        core_axis_name="c", subcore_axis_name="s", num_cores=NC, num_subcores=NS
    )

    @functools.partial(
        pl.kernel,
        out_type=jax.ShapeDtypeStruct((NP * CH,), jnp.float32),
        mesh=mesh,
        scratch_types=[
            pltpu.VMEM((VB * G,), jnp.int32),
            pltpu.VMEM((NH * VB * G,), jnp.int32),
            pltpu.VMEM((NH * VB * G,), jnp.float32),
            pltpu.VMEM((NH * VB * CH,), jnp.float32),
            pltpu.VMEM((VB * G,), jnp.int32),
            pltpu.VMEM((VB * G,), jnp.int32),
            pltpu.VMEM((VB * G, CH), jnp.bfloat16),
            pltpu.VMEM((VB * G, CH), jnp.bfloat16),
            pltpu.VMEM((NH * VB * CH,), jnp.float32),
            pltpu.SemaphoreType.DMA,
            pltpu.SemaphoreType.DMA,
        ],
        compiler_params=pltpu.CompilerParams(
            needs_layout_passes=False, use_tc_tiling_on_sc=False
        ),
    )
    def k(table_h, idx_h, w_h, c32_h, offs_h, out_h,
          offs_v, sidx_v, sw_v, sc32_v, row0_v, row1_v, gath0_v, gath1_v,
          out_v, sem_g0, sem_g1):
        wid = lax.axis_index("s") * NC + lax.axis_index("c")
        pltpu.sync_copy(offs_h, offs_v)
        base0 = wid * NB
        slots = ((row0_v, gath0_v, sem_g0), (row1_v, gath1_v, sem_g1))

        def rows(sb, par):
            row_v = slots[par][0]
            for s in range(VB * G // 16):
                row_v[pl.ds(s * 16, 16)] = (
                    sidx_v[pl.ds(sb * (VB * G) + s * 16, 16)] * RA
                    + offs_v[pl.ds(s * 16, 16)]
                )

        def start_gather(par):
            row_v, gath_v, sem = slots[par]
            return pltpu.async_copy(table_h.at[row_v], gath_v, sem)

        def wait_gather(par):
            row_v, gath_v, sem = slots[par]
            pltpu.make_async_copy(table_h.at[row_v], gath_v, sem).wait()

        def compute(sb, par):
            gath_v = slots[par][1]

            def vert(p, c2):
                cbase = sb * (VB * CH) + p * CH
                acc0 = sc32_v[pl.ds(cbase, 16)]
                acc1 = sc32_v[pl.ds(cbase + 16, 16)]
                wbase = sb * (VB * G) + p * G
                for i in range(G):
                    wi = plsc.load_gather(
                        sw_v, [jnp.full((16,), wbase + i, jnp.int32)]
                    )
                    pos = p * G + i
                    g0, g1 = plsc.unpack(
                        gath_v[pos, pl.ds(0, CH)],
                        format=plsc.PackFormat.INTERLEAVED,
                    )
                    acc0 = acc0 + wi * g0
                    acc1 = acc1 + wi * g1
                obase = sb * (VB * CH) + p * CH
                out_v[pl.ds(obase, 16)] = jnp.maximum(acc0, 0.0)
                out_v[pl.ds(obase + 16, 16)] = jnp.maximum(acc1, 0.0)
                return c2

            lax.fori_loop(0, VB, vert, 0)

        def half(h, carry):
            hbase = base0 + h * NH
            pltpu.sync_copy(
                idx_h.at[pl.ds(hbase * (VB * G), NH * VB * G)], sidx_v
            )
            pltpu.sync_copy(
                w_h.at[pl.ds(hbase * (VB * G), NH * VB * G)], sw_v
            )
            pltpu.sync_copy(
                c32_h.at[pl.ds(hbase * (VB * CH), NH * VB * CH)], sc32_v
            )
            rows(0, 0)
            start_gather(0)

            def pair(t, c2):
                sb0 = 2 * t
                rows(sb0 + 1, 1)
                start_gather(1)
                wait_gather(0)
                compute(sb0, 0)

                @pl.when(t < NH // 2 - 1)
                def _():
                    rows(sb0 + 2, 0)
                    start_gather(0)

                wait_gather(1)
                compute(sb0 + 1, 1)
                return c2

            lax.fori_loop(0, NH // 2, pair, 0)
            pltpu.sync_copy(
                out_v, out_h.at[pl.ds(hbase * (VB * CH), NH * VB * CH)]
            )
            return carry

        lax.fori_loop(0, 2, half, 0)

    return k(table, idx_flat, w_flat, c32_flat, offs)


@jax.jit
def kernel(mesh_signal, bary_coordinates, neighbor_weights, self_weights, bias):
    rolled = jnp.stack(
        [jnp.roll(neighbor_weights, -2 * oi, axis=2) for oi in range(NROT)], axis=0
    )
    wroll = (
        rolled.transpose(2, 3, 0, 1, 4)
        .reshape(RA, 2, CH // 2, F)
        .transpose(0, 2, 1, 3)
        .reshape(RA * CH, F)
        .T
    )
    wc = jnp.tile(self_weights[:, 0, :].T, (1, NROT))
    bias32 = jnp.tile(bias, NROT)[None, :]

    ms_pad = jnp.pad(mesh_signal, ((0, NP - N), (0, 0)))

    idx = bary_coordinates[..., 0].astype(jnp.int32).reshape(N, G)
    w = bary_coordinates[..., 1].reshape(N, G)
    idx_pad = jnp.pad(idx, ((0, NP - N), (0, 0))).reshape(NP // VB, VB * G)
    w_pad = jnp.pad(w, ((0, NP - N), (0, 0))).reshape(NP // VB, VB * G)
    offs = jnp.tile(jnp.arange(G, dtype=jnp.int32) // 3, VB)

    qroll, c32 = _tc_project(ms_pad, wroll, wc, bias32)
    table = qroll.reshape(NP * RA, CH)

    out = _sc_gather_accum(
        table,
        idx_pad.reshape(-1),
        w_pad.reshape(-1),
        c32.reshape(-1),
        offs,
    )

    return out.reshape(NP, NROT, T)[:N]
